# baseline (speedup 1.0000x reference)
import functools

import jax
import jax.numpy as jnp
from jax import lax
from jax.experimental import pallas as pl
from jax.experimental.pallas import tpu as pltpu
from jax.experimental.pallas import tpu_sc as plsc

B, S, D = 4, 8192, 2048
L = 16
NW = 16
CHUNK = (B * S) // NW
UNROLL = 8
SPLIT = NW // B


def _pool_body(mask_hbm, hs_hbm, out_hbm, mchunk_v, stage_v, red_v, idx_v,
               row_v, shared_sp, sem):
    s = lax.axis_index("s")

    pltpu.sync_copy(mask_hbm.at[pl.ds(s * CHUNK, CHUNK)], mchunk_v)

    def body(i, acc):
        base = i * (L * UNROLL)
        for u in range(UNROLL):
            acc = acc + mchunk_v[pl.ds(base + u * L, L)]
        return acc

    acc = lax.fori_loop(
        0, CHUNK // (L * UNROLL), body, jnp.zeros((L,), jnp.int32)
    )
    stage_v[...] = acc
    pltpu.sync_copy(stage_v, shared_sp.at[pl.ds(s * L, L)])
    plsc.subcore_barrier()

    @pl.when(s < B)
    def _():
        pltpu.sync_copy(shared_sp.at[pl.ds(s * SPLIT * L, SPLIT * L)], red_v)
        comb = red_v[pl.ds(0, L)]
        for j in range(1, SPLIT):
            comb = comb + red_v[pl.ds(j * L, L)]
        seq_len = comb[0]
        for k in range(1, L):
            seq_len = seq_len + comb[k]
        idx = s * S + seq_len - 1
        idx_v[...] = jnp.full((L,), idx, jnp.int32)
        pltpu.async_copy(hs_hbm.at[idx_v.at[pl.ds(0, 1)]], row_v, sem).wait()
        pltpu.sync_copy(row_v, out_hbm.at[pl.ds(s, 1)])


_pooled = functools.partial(
    pl.kernel,
    out_type=jax.ShapeDtypeStruct((B, D), jnp.float32),
    mesh=plsc.VectorSubcoreMesh(
        core_axis_name="c", subcore_axis_name="s", num_cores=1
    ),
    scratch_types=[
        pltpu.VMEM((CHUNK,), jnp.int32),
        pltpu.VMEM((L,), jnp.int32),
        pltpu.VMEM((SPLIT * L,), jnp.int32),
        pltpu.VMEM((L,), jnp.int32),
        pltpu.VMEM((1, D), jnp.float32),
        pltpu.VMEM_SHARED((NW * L,), jnp.int32),
        pltpu.SemaphoreType.DMA,
    ],
)(_pool_body)


def kernel(hidden_states, attention_mask):
    hs2 = hidden_states.reshape(B * S, D)
    mask = attention_mask.astype(jnp.int32).reshape(B * S)
    return _pooled(mask, hs2)


# speedup vs baseline: 1.0096x; 1.0096x over previous
import functools

import jax
import jax.numpy as jnp
from jax import lax
from jax.experimental import pallas as pl
from jax.experimental.pallas import tpu as pltpu
from jax.experimental.pallas import tpu_sc as plsc

B, S, D = 4, 8192, 2048
L = 16
UNROLL = 16
HALF = S // 2


def _pool_body(mask_hbm, hs_hbm, out_hbm, mrow_v, idx_v, row_v, sem, sem2):
    c = lax.axis_index("c")
    s = lax.axis_index("s")

    @pl.when((c == 0) & (s < B))
    def _():
        cp1 = pltpu.async_copy(
            mask_hbm.at[s, pl.ds(0, HALF)], mrow_v.at[pl.ds(0, HALF)], sem
        )
        cp2 = pltpu.async_copy(
            mask_hbm.at[s, pl.ds(HALF, HALF)],
            mrow_v.at[pl.ds(HALF, HALF)],
            sem2,
        )

        def body(i, acc):
            base = i * (L * UNROLL)
            for u in range(UNROLL):
                acc = acc + mrow_v[pl.ds(base + u * L, L)]
            return acc

        cp1.wait()
        acc = lax.fori_loop(
            0, HALF // (L * UNROLL), body, jnp.zeros((L,), jnp.int32)
        )
        cp2.wait()
        acc = lax.fori_loop(
            HALF // (L * UNROLL), S // (L * UNROLL), body, acc
        )
        seq_len = acc[0]
        for k in range(1, L):
            seq_len = seq_len + acc[k]
        idx = s * S + seq_len - 1
        idx_v[...] = jnp.full((L,), idx, jnp.int32)

        pltpu.async_copy(hs_hbm.at[idx_v.at[pl.ds(0, 1)]], row_v, sem).wait()
        pltpu.sync_copy(row_v, out_hbm.at[pl.ds(s, 1)])


_pooled = functools.partial(
    pl.kernel,
    out_type=jax.ShapeDtypeStruct((B, D), jnp.float32),
    mesh=plsc.VectorSubcoreMesh(
        core_axis_name="c", subcore_axis_name="s", num_cores=1
    ),
    scratch_types=[
        pltpu.VMEM((S,), jnp.int32),
        pltpu.VMEM((L,), jnp.int32),
        pltpu.VMEM((1, D), jnp.float32),
        pltpu.SemaphoreType.DMA,
        pltpu.SemaphoreType.DMA,
    ],
)(_pool_body)


def kernel(hidden_states, attention_mask):
    hs2 = hidden_states.reshape(B * S, D)
    mask = attention_mask.astype(jnp.int32)
    return _pooled(mask, hs2)
